# Initial kernel scaffold; baseline (speedup 1.0000x reference)
import jax
import jax.numpy as jnp
from jax.experimental import pallas as pl


def kernel(x, codebook):
    raise NotImplementedError("write your pallas kernel here")


# speedup vs baseline: 1.0062x; 1.0062x over previous
import functools

import jax
import jax.numpy as jnp
from jax import lax
from jax.experimental import pallas as pl
from jax.experimental.pallas import tpu as pltpu
from jax.experimental.pallas import tpu_sc as plsc

NB = 8192
CD = 256
NTOK = 4096
BR = 1024
BC = 1024
NI = NTOK // BR
NJ = NB // BC

NW = 32
BPW = NTOK // NW
LANES = 16


def _dist_argmax_body(x_ref, cb_ref, idx_ref, commit_ref, runmax_ref, runidx_ref):
    j = pl.program_id(1)

    @pl.when(j == 0)
    def _init():
        runmax_ref[...] = jnp.full((BR,), -jnp.inf, jnp.float32)
        runidx_ref[...] = jnp.zeros((BR,), jnp.int32)

    x = x_ref[...]
    cb = cb_ref[...]
    dots = lax.dot_general(x, cb, (((1,), (1,)), ((), ())),
                           preferred_element_type=jnp.float32)
    cnorm = jnp.sum(cb * cb, axis=1)
    logits = 2.0 * dots - cnorm[None, :]
    bmax = jnp.max(logits, axis=1)
    lidx = lax.broadcasted_iota(jnp.int32, (BR, BC), 1)
    barg = jnp.min(jnp.where(logits == bmax[:, None], lidx, BC), axis=1) + j * BC
    better = bmax > runmax_ref[...]
    newmax = jnp.where(better, bmax, runmax_ref[...])
    newidx = jnp.where(better, barg, runidx_ref[...])
    runmax_ref[...] = newmax
    runidx_ref[...] = newidx

    @pl.when(j == NJ - 1)
    def _finish():
        idx_ref[...] = newidx
        rowsq = jnp.sum(x * x, axis=1)
        part = jnp.sum(rowsq - newmax)

        @pl.when(pl.program_id(0) == 0)
        def _zero():
            commit_ref[0] = 0.0

        commit_ref[0] += part


def _dist_argmax(x_flat, codebook):
    return pl.pallas_call(
        _dist_argmax_body,
        grid=(NI, NJ),
        in_specs=[
            pl.BlockSpec((BR, CD), lambda i, j: (i, 0)),
            pl.BlockSpec((BC, CD), lambda i, j: (j, 0)),
        ],
        out_specs=[
            pl.BlockSpec((BR,), lambda i, j: (i,)),
            pl.BlockSpec(memory_space=pltpu.SMEM),
        ],
        out_shape=[
            jax.ShapeDtypeStruct((NTOK,), jnp.int32),
            jax.ShapeDtypeStruct((1,), jnp.float32),
        ],
        scratch_shapes=[
            pltpu.VMEM((BR,), jnp.float32),
            pltpu.VMEM((BR,), jnp.int32),
        ],
    )(x_flat, codebook)


def _sc_body(cb_hbm, idx_hbm, xd_hbm, counts_hbm, idx_v, rows_v, cnt_v, sem):
    wid = lax.axis_index("s") * 2 + lax.axis_index("c")
    base = wid * BPW
    pltpu.sync_copy(idx_hbm.at[pl.ds(base, BPW)], idx_v)
    cp = pltpu.async_copy(cb_hbm.at[idx_v], rows_v, sem)

    def _zero_block(i, carry):
        cnt_v[pl.ds(i * LANES, LANES)] = jnp.zeros((LANES,), jnp.float32)
        return carry

    lax.fori_loop(0, NB // LANES, _zero_block, 0)

    ones = jnp.ones((LANES,), jnp.float32)
    lane = lax.iota(jnp.int32, LANES)
    for v in range(BPW // LANES):
        iv = idx_v[pl.ds(v * LANES, LANES)]
        for k in range(LANES):
            plsc.addupdate_scatter(cnt_v, [iv], ones, mask=lane == k)

    cp.wait()
    pltpu.sync_copy(rows_v, xd_hbm.at[pl.ds(base, BPW)])
    pltpu.sync_copy(cnt_v, counts_hbm.at[wid])


def _sc_gather_hist(codebook, code_idx):
    call = pl.kernel(
        _sc_body,
        mesh=plsc.VectorSubcoreMesh(core_axis_name="c", subcore_axis_name="s"),
        out_type=[
            jax.ShapeDtypeStruct((NTOK, CD), jnp.float32),
            jax.ShapeDtypeStruct((NW, NB), jnp.float32),
        ],
        scratch_types=[
            pltpu.VMEM((BPW,), jnp.int32),
            pltpu.VMEM((BPW, CD), jnp.float32),
            pltpu.VMEM((NB,), jnp.float32),
            pltpu.SemaphoreType.DMA,
        ],
        compiler_params=pltpu.CompilerParams(needs_layout_passes=False),
    )
    return call(codebook, code_idx)


def _perplexity_body(pc_ref, out_ref):
    counts = jnp.sum(pc_ref[...], axis=0)
    p = counts * (1.0 / NTOK)
    ent = jnp.sum(p * jnp.log(p + 1e-7))
    out_ref[0] = jnp.exp(-ent)


def _perplexity(partial_counts):
    return pl.pallas_call(
        _perplexity_body,
        out_specs=pl.BlockSpec(memory_space=pltpu.SMEM),
        out_shape=jax.ShapeDtypeStruct((1,), jnp.float32),
    )(partial_counts)


def kernel(x, codebook):
    N, C, T = x.shape
    x_flat = jnp.transpose(x, (0, 2, 1)).reshape(N * T, C)
    code_idx, commit_sum = _dist_argmax(x_flat, codebook)
    x_d, partial_counts = _sc_gather_hist(codebook, code_idx)
    perp = _perplexity(partial_counts)
    commit_loss = commit_sum[0] * (1.0 / (NTOK * CD))
    x_d_out = jnp.transpose(x_d.reshape(N, T, C), (0, 2, 1))
    return (x_d_out, commit_loss, perp[0])


# speedup vs baseline: 1.2855x; 1.2776x over previous
import functools

import jax
import jax.numpy as jnp
from jax import lax
from jax.experimental import pallas as pl
from jax.experimental.pallas import tpu as pltpu
from jax.experimental.pallas import tpu_sc as plsc

NB = 8192
CD = 256
NTOK = 4096
BR = 1024
BC = 1024
NI = NTOK // BR
NJ = NB // BC

NW = 32
BPW = NTOK // NW
LANES = 16


def _dist_argmax_body(x_ref, cb_ref, idx_ref, commit_ref, cnorm_ref,
                      rowsq_ref, runmin_ref, runidx_ref):
    i = pl.program_id(0)
    j = pl.program_id(1)

    @pl.when(i == 0)
    def _cnorm():
        cb = cb_ref[...]
        cnorm_ref[pl.ds(j * BC, BC), :] = jnp.sum(cb * cb, axis=1,
                                                  keepdims=True)

    @pl.when(j == 0)
    def _init():
        x = x_ref[...]
        rowsq_ref[...] = jnp.sum(x * x, axis=1)[None, :]
        runmin_ref[...] = jnp.full((BR,), jnp.inf, jnp.float32)
        runidx_ref[...] = jnp.zeros((BR,), jnp.int32)

    x = x_ref[...]
    cb = cb_ref[...]
    dots = lax.dot_general(cb, x, (((1,), (1,)), ((), ())),
                           preferred_element_type=jnp.float32)
    dist = (rowsq_ref[...] - 2.0 * dots) + cnorm_ref[pl.ds(j * BC, BC), :]
    bmin = jnp.min(dist, axis=0)
    sidx = lax.broadcasted_iota(jnp.int32, (BC, BR), 0)
    barg = jnp.min(jnp.where(dist == bmin[None, :], sidx, BC), axis=0) + j * BC
    better = bmin < runmin_ref[...]
    newmin = jnp.where(better, bmin, runmin_ref[...])
    newidx = jnp.where(better, barg, runidx_ref[...])
    runmin_ref[...] = newmin
    runidx_ref[...] = newidx

    @pl.when(j == NJ - 1)
    def _finish():
        idx_ref[...] = newidx
        part = jnp.sum(newmin)

        @pl.when(i == 0)
        def _zero():
            commit_ref[0] = 0.0

        commit_ref[0] += part


def _dist_argmax(x_flat, codebook):
    return pl.pallas_call(
        _dist_argmax_body,
        grid=(NI, NJ),
        in_specs=[
            pl.BlockSpec((BR, CD), lambda i, j: (i, 0)),
            pl.BlockSpec((BC, CD), lambda i, j: (j, 0)),
        ],
        out_specs=[
            pl.BlockSpec((BR,), lambda i, j: (i,)),
            pl.BlockSpec(memory_space=pltpu.SMEM),
        ],
        out_shape=[
            jax.ShapeDtypeStruct((NTOK,), jnp.int32),
            jax.ShapeDtypeStruct((1,), jnp.float32),
        ],
        scratch_shapes=[
            pltpu.VMEM((NB, 1), jnp.float32),
            pltpu.VMEM((1, BR), jnp.float32),
            pltpu.VMEM((BR,), jnp.float32),
            pltpu.VMEM((BR,), jnp.int32),
        ],
    )(x_flat, codebook)


def _sc_body(cb_hbm, idx_hbm, xd_hbm, counts_hbm, idx_v, rows_v, cnt_v, sem):
    wid = lax.axis_index("s") * 2 + lax.axis_index("c")
    base = wid * BPW
    pltpu.sync_copy(idx_hbm.at[pl.ds(base, BPW)], idx_v)
    cp = pltpu.async_copy(cb_hbm.at[idx_v], rows_v, sem)

    def _zero_block(i, carry):
        cnt_v[pl.ds(i * LANES, LANES)] = jnp.zeros((LANES,), jnp.float32)
        return carry

    lax.fori_loop(0, NB // LANES, _zero_block, 0)

    ones = jnp.ones((LANES,), jnp.float32)
    lane = lax.iota(jnp.int32, LANES)
    for v in range(BPW // LANES):
        iv = idx_v[pl.ds(v * LANES, LANES)]
        for k in range(LANES):
            plsc.addupdate_scatter(cnt_v, [iv], ones, mask=lane == k)

    cp.wait()
    pltpu.sync_copy(rows_v, xd_hbm.at[pl.ds(base, BPW)])
    pltpu.sync_copy(cnt_v, counts_hbm.at[wid])


def _sc_gather_hist(codebook, code_idx):
    call = pl.kernel(
        _sc_body,
        mesh=plsc.VectorSubcoreMesh(core_axis_name="c", subcore_axis_name="s"),
        out_type=[
            jax.ShapeDtypeStruct((NTOK, CD), jnp.float32),
            jax.ShapeDtypeStruct((NW, NB), jnp.float32),
        ],
        scratch_types=[
            pltpu.VMEM((BPW,), jnp.int32),
            pltpu.VMEM((BPW, CD), jnp.float32),
            pltpu.VMEM((NB,), jnp.float32),
            pltpu.SemaphoreType.DMA,
        ],
        compiler_params=pltpu.CompilerParams(needs_layout_passes=False),
    )
    return call(codebook, code_idx)


def _perplexity_body(pc_ref, out_ref):
    counts = jnp.sum(pc_ref[...], axis=0)
    p = counts * (1.0 / NTOK)
    ent = jnp.sum(p * jnp.log(p + 1e-7))
    out_ref[0] = jnp.exp(-ent)


def _perplexity(partial_counts):
    return pl.pallas_call(
        _perplexity_body,
        out_specs=pl.BlockSpec(memory_space=pltpu.SMEM),
        out_shape=jax.ShapeDtypeStruct((1,), jnp.float32),
    )(partial_counts)


def kernel(x, codebook):
    N, C, T = x.shape
    x_flat = jnp.transpose(x, (0, 2, 1)).reshape(N * T, C)
    code_idx, commit_sum = _dist_argmax(x_flat, codebook)
    x_d, partial_counts = _sc_gather_hist(codebook, code_idx)
    perp = _perplexity(partial_counts)
    commit_loss = commit_sum[0] * (1.0 / (NTOK * CD))
    x_d_out = jnp.transpose(x_d.reshape(N, T, C), (0, 2, 1))
    return (x_d_out, commit_loss, perp[0])


# speedup vs baseline: 1.3383x; 1.0410x over previous
import functools

import jax
import jax.numpy as jnp
from jax import lax
from jax.experimental import pallas as pl
from jax.experimental.pallas import tpu as pltpu
from jax.experimental.pallas import tpu_sc as plsc

NB = 8192
CD = 256
NTOK = 4096
BR = 1024
BC = 1024
NI = NTOK // BR
NJ = NB // BC

NW = 32
BPW = NTOK // NW
LANES = 16


NBATCH = BR // 256


def _dist_argmax_body(x_ref, cb_ref, idx_ref, commit_ref, cnorm_ref,
                      rowsq_ref, runmin_ref, runidx_ref):
    i = pl.program_id(0)
    j = pl.program_id(1)

    @pl.when(i == 0)
    def _cnorm():
        cb = cb_ref[...]
        cnorm_ref[pl.ds(j * BC, BC), :] = jnp.sum(cb * cb, axis=1,
                                                  keepdims=True)

    @pl.when(j == 0)
    def _init():
        rowsq_ref[...] = jnp.concatenate(
            [jnp.sum(x_ref[n] * x_ref[n], axis=0)[None, :]
             for n in range(NBATCH)], axis=1)
        runmin_ref[...] = jnp.full((BR,), jnp.inf, jnp.float32)
        runidx_ref[...] = jnp.zeros((BR,), jnp.int32)

    cb = cb_ref[...]
    dots = jnp.concatenate(
        [lax.dot_general(cb, x_ref[n], (((1,), (0,)), ((), ())),
                         preferred_element_type=jnp.float32)
         for n in range(NBATCH)], axis=1)
    dist = (rowsq_ref[...] - 2.0 * dots) + cnorm_ref[pl.ds(j * BC, BC), :]
    bmin = jnp.min(dist, axis=0)
    sidx = lax.broadcasted_iota(jnp.int32, (BC, BR), 0)
    barg = jnp.min(jnp.where(dist == bmin[None, :], sidx, BC), axis=0) + j * BC
    better = bmin < runmin_ref[...]
    newmin = jnp.where(better, bmin, runmin_ref[...])
    newidx = jnp.where(better, barg, runidx_ref[...])
    runmin_ref[...] = newmin
    runidx_ref[...] = newidx

    @pl.when(j == NJ - 1)
    def _finish():
        idx_ref[...] = newidx
        part = jnp.sum(newmin)

        @pl.when(i == 0)
        def _zero():
            commit_ref[0] = 0.0

        commit_ref[0] += part


def _dist_argmax(x, codebook):
    return pl.pallas_call(
        _dist_argmax_body,
        grid=(NI, NJ),
        in_specs=[
            pl.BlockSpec((NBATCH, CD, 256), lambda i, j: (i, 0, 0)),
            pl.BlockSpec((BC, CD), lambda i, j: (j, 0)),
        ],
        out_specs=[
            pl.BlockSpec((BR,), lambda i, j: (i,)),
            pl.BlockSpec(memory_space=pltpu.SMEM),
        ],
        out_shape=[
            jax.ShapeDtypeStruct((NTOK,), jnp.int32),
            jax.ShapeDtypeStruct((1,), jnp.float32),
        ],
        scratch_shapes=[
            pltpu.VMEM((NB, 1), jnp.float32),
            pltpu.VMEM((1, BR), jnp.float32),
            pltpu.VMEM((BR,), jnp.float32),
            pltpu.VMEM((BR,), jnp.int32),
        ],
    )(x, codebook)


def _sc_body(cb_hbm, idx_hbm, xd_hbm, counts_hbm, idx_v, rows_v, cnt_v, sem):
    wid = lax.axis_index("s") * 2 + lax.axis_index("c")
    base = wid * BPW
    pltpu.sync_copy(idx_hbm.at[pl.ds(base, BPW)], idx_v)
    cp = pltpu.async_copy(cb_hbm.at[idx_v], rows_v, sem)

    def _zero_block(i, carry):
        cnt_v[pl.ds(i * LANES, LANES)] = jnp.zeros((LANES,), jnp.float32)
        return carry

    lax.fori_loop(0, NB // LANES, _zero_block, 0)

    ones = jnp.ones((LANES,), jnp.float32)
    lane = lax.iota(jnp.int32, LANES)
    for v in range(BPW // LANES):
        iv = idx_v[pl.ds(v * LANES, LANES)]
        for k in range(LANES):
            plsc.addupdate_scatter(cnt_v, [iv], ones, mask=lane == k)

    cp.wait()
    pltpu.sync_copy(rows_v, xd_hbm.at[pl.ds(base, BPW)])
    pltpu.sync_copy(cnt_v, counts_hbm.at[wid])


def _sc_gather_hist(codebook, code_idx):
    call = pl.kernel(
        _sc_body,
        mesh=plsc.VectorSubcoreMesh(core_axis_name="c", subcore_axis_name="s"),
        out_type=[
            jax.ShapeDtypeStruct((NTOK, CD), jnp.float32),
            jax.ShapeDtypeStruct((NW, NB), jnp.float32),
        ],
        scratch_types=[
            pltpu.VMEM((BPW,), jnp.int32),
            pltpu.VMEM((BPW, CD), jnp.float32),
            pltpu.VMEM((NB,), jnp.float32),
            pltpu.SemaphoreType.DMA,
        ],
        compiler_params=pltpu.CompilerParams(needs_layout_passes=False),
    )
    return call(codebook, code_idx)


def _finalize_body(xd_ref, pc_ref, out_ref, perp_ref):
    out_ref[0] = xd_ref[...].T

    @pl.when(pl.program_id(0) == 0)
    def _perp():
        counts = jnp.sum(pc_ref[...], axis=0)
        p = counts * (1.0 / NTOK)
        ent = jnp.sum(p * jnp.log(p + 1e-7))
        perp_ref[0] = jnp.exp(-ent)


def _finalize(x_d, partial_counts, nbatches):
    return pl.pallas_call(
        _finalize_body,
        grid=(nbatches,),
        in_specs=[
            pl.BlockSpec((256, CD), lambda b: (b, 0)),
            pl.BlockSpec((NW, NB), lambda b: (0, 0)),
        ],
        out_specs=[
            pl.BlockSpec((1, CD, 256), lambda b: (b, 0, 0)),
            pl.BlockSpec(memory_space=pltpu.SMEM),
        ],
        out_shape=[
            jax.ShapeDtypeStruct((nbatches, CD, 256), jnp.float32),
            jax.ShapeDtypeStruct((1,), jnp.float32),
        ],
    )(x_d, partial_counts)


def kernel(x, codebook):
    N, C, T = x.shape
    code_idx, commit_sum = _dist_argmax(x, codebook)
    x_d, partial_counts = _sc_gather_hist(codebook, code_idx)
    x_d_out, perp = _finalize(x_d, partial_counts, N)
    commit_loss = commit_sum[0] * (1.0 / (NTOK * CD))
    return (x_d_out, commit_loss, perp[0])


# speedup vs baseline: 1.4762x; 1.1031x over previous
import functools

import jax
import jax.numpy as jnp
from jax import lax
from jax.experimental import pallas as pl
from jax.experimental.pallas import tpu as pltpu
from jax.experimental.pallas import tpu_sc as plsc

NB = 8192
CD = 256
NTOK = 4096
BR = 1024
BC = 2048
NI = NTOK // BR
NJ = NB // BC

NW = 32
BPW = NTOK // NW
LANES = 16


NBATCH = BR // 256


def _dist_argmax_body(x_ref, cb_ref, idx_ref, commit_ref, cnorm_ref,
                      rowsq_ref, x2_ref, iotaf_ref, runmin_ref, runidx_ref):
    i = pl.program_id(0)
    j = pl.program_id(1)

    @pl.when(i == 0)
    def _cnorm():
        cb = cb_ref[...]
        cnorm_ref[pl.ds(j * BC, BC), :] = jnp.sum(cb * cb, axis=1,
                                                  keepdims=True)

    @pl.when(j == 0)
    def _init():
        rowsq_ref[...] = jnp.concatenate(
            [jnp.sum(x_ref[n] * x_ref[n], axis=0)[None, :]
             for n in range(NBATCH)], axis=1)
        x2_ref[...] = x_ref[...] + x_ref[...]
        runmin_ref[...] = jnp.full((BR,), jnp.inf, jnp.float32)
        runidx_ref[...] = jnp.zeros((BR,), jnp.int32)

        @pl.when(i == 0)
        def _iota():
            iotaf_ref[...] = lax.broadcasted_iota(
                jnp.int32, (BC, 1), 0).astype(jnp.float32)

    cb = cb_ref[...]
    dots2 = jnp.concatenate(
        [lax.dot_general(cb, x2_ref[n], (((1,), (0,)), ((), ())),
                         preferred_element_type=jnp.float32)
         for n in range(NBATCH)], axis=1)
    dist = (rowsq_ref[...] - dots2) + cnorm_ref[pl.ds(j * BC, BC), :]
    bmin = jnp.min(dist, axis=0)
    cand = jnp.where(dist == bmin[None, :], iotaf_ref[...], float(BC))
    barg = jnp.min(cand, axis=0).astype(jnp.int32) + j * BC
    better = bmin < runmin_ref[...]
    newmin = jnp.where(better, bmin, runmin_ref[...])
    newidx = jnp.where(better, barg, runidx_ref[...])
    runmin_ref[...] = newmin
    runidx_ref[...] = newidx

    @pl.when(j == NJ - 1)
    def _finish():
        idx_ref[...] = newidx
        part = jnp.sum(newmin)

        @pl.when(i == 0)
        def _zero():
            commit_ref[0] = 0.0

        commit_ref[0] += part


def _dist_argmax(x, codebook):
    return pl.pallas_call(
        _dist_argmax_body,
        grid=(NI, NJ),
        in_specs=[
            pl.BlockSpec((NBATCH, CD, 256), lambda i, j: (i, 0, 0)),
            pl.BlockSpec((BC, CD), lambda i, j: (j, 0)),
        ],
        out_specs=[
            pl.BlockSpec((BR,), lambda i, j: (i,)),
            pl.BlockSpec(memory_space=pltpu.SMEM),
        ],
        out_shape=[
            jax.ShapeDtypeStruct((NTOK,), jnp.int32),
            jax.ShapeDtypeStruct((1,), jnp.float32),
        ],
        scratch_shapes=[
            pltpu.VMEM((NB, 1), jnp.float32),
            pltpu.VMEM((1, BR), jnp.float32),
            pltpu.VMEM((NBATCH, CD, 256), jnp.float32),
            pltpu.VMEM((BC, 1), jnp.float32),
            pltpu.VMEM((BR,), jnp.float32),
            pltpu.VMEM((BR,), jnp.int32),
        ],
    )(x, codebook)


def _sc_body(cb_hbm, idx_hbm, xd_hbm, counts_hbm, idx_v, rows_v, cnt_v, sem):
    wid = lax.axis_index("s") * 2 + lax.axis_index("c")
    base = wid * BPW
    pltpu.sync_copy(idx_hbm.at[pl.ds(base, BPW)], idx_v)
    cp = pltpu.async_copy(cb_hbm.at[idx_v], rows_v, sem)

    zeros = jnp.zeros((LANES,), jnp.float32)

    def _zero_block(b, carry):
        for u in range(8):
            cnt_v[pl.ds((b * 8 + u) * LANES, LANES)] = zeros
        return carry

    lax.fori_loop(0, NB // (8 * LANES), _zero_block, 0)

    ones = jnp.ones((LANES,), jnp.float32)
    lane = lax.iota(jnp.int32, LANES)
    for v in range(BPW // LANES):
        iv = idx_v[pl.ds(v * LANES, LANES)]
        for k in range(LANES):
            plsc.addupdate_scatter(cnt_v, [iv], ones, mask=lane == k)

    cp.wait()
    pltpu.sync_copy(rows_v, xd_hbm.at[pl.ds(base, BPW)])
    pltpu.sync_copy(cnt_v, counts_hbm.at[wid])


def _sc_gather_hist(codebook, code_idx):
    call = pl.kernel(
        _sc_body,
        mesh=plsc.VectorSubcoreMesh(core_axis_name="c", subcore_axis_name="s"),
        out_type=[
            jax.ShapeDtypeStruct((NTOK, CD), jnp.float32),
            jax.ShapeDtypeStruct((NW, NB), jnp.float32),
        ],
        scratch_types=[
            pltpu.VMEM((BPW,), jnp.int32),
            pltpu.VMEM((BPW, CD), jnp.float32),
            pltpu.VMEM((NB,), jnp.float32),
            pltpu.SemaphoreType.DMA,
        ],
        compiler_params=pltpu.CompilerParams(needs_layout_passes=False),
    )
    return call(codebook, code_idx)


def _finalize_body(xd_ref, pc_ref, out_ref, perp_ref):
    out_ref[0] = xd_ref[...].T

    @pl.when(pl.program_id(0) == 0)
    def _perp():
        counts = jnp.sum(pc_ref[...], axis=0)
        p = counts * (1.0 / NTOK)
        ent = jnp.sum(p * jnp.log(p + 1e-7))
        perp_ref[0] = jnp.exp(-ent)


def _finalize(x_d, partial_counts, nbatches):
    return pl.pallas_call(
        _finalize_body,
        grid=(nbatches,),
        in_specs=[
            pl.BlockSpec((256, CD), lambda b: (b, 0)),
            pl.BlockSpec((NW, NB), lambda b: (0, 0)),
        ],
        out_specs=[
            pl.BlockSpec((1, CD, 256), lambda b: (b, 0, 0)),
            pl.BlockSpec(memory_space=pltpu.SMEM),
        ],
        out_shape=[
            jax.ShapeDtypeStruct((nbatches, CD, 256), jnp.float32),
            jax.ShapeDtypeStruct((1,), jnp.float32),
        ],
    )(x_d, partial_counts)


def kernel(x, codebook):
    N, C, T = x.shape
    code_idx, commit_sum = _dist_argmax(x, codebook)
    x_d, partial_counts = _sc_gather_hist(codebook, code_idx)
    x_d_out, perp = _finalize(x_d, partial_counts, N)
    commit_loss = commit_sum[0] * (1.0 / (NTOK * CD))
    return (x_d_out, commit_loss, perp[0])
